# baseline (speedup 1.0000x reference)
import functools

import jax
import jax.numpy as jnp
from jax import lax
from jax.experimental import pallas as pl
from jax.experimental.pallas import tpu as pltpu
from jax.experimental.pallas import tpu_sc as plsc

FEAT_H = 32
FEAT_W = 106
NUM_A = 36
NCH = 7
HW = FEAT_H * FEAT_W
ROWW = NUM_A * NCH
PLANE = HW * ROWW
BATCH = 4
GROUP_W = 4 * ROWW
CHUNK_ROWS = 8
CHUNK_W = CHUNK_ROWS * ROWW
NCHUNKS = HW // CHUNK_ROWS
NWORKERS = 32
MAXK = (NCHUNKS + NWORKERS - 1) // NWORKERS
NSTEPS = BATCH * MAXK
VPC = CHUNK_W // 16


def _sc_denorm(tabs, rowloc, xflat):
    mesh = plsc.VectorSubcoreMesh(core_axis_name="c", subcore_axis_name="s")

    @functools.partial(
        pl.kernel,
        out_type=jax.ShapeDtypeStruct((BATCH * PLANE,), jnp.float32),
        mesh=mesh,
        scratch_types=[
            pltpu.VMEM((5 * GROUP_W,), jnp.float32),
            pltpu.VMEM((GROUP_W,), jnp.float32),
            pltpu.VMEM((MAXK * CHUNK_W,), jnp.float32),
            pltpu.VMEM((CHUNK_W,), jnp.float32),
            pltpu.VMEM((CHUNK_W,), jnp.float32),
            pltpu.VMEM((CHUNK_W,), jnp.float32),
            pltpu.VMEM((CHUNK_W,), jnp.float32),
            pltpu.SemaphoreType.DMA,
            pltpu.SemaphoreType.DMA,
            pltpu.SemaphoreType.DMA,
            pltpu.SemaphoreType.DMA,
        ],
    )
    def body(tabs_hbm, rowloc_hbm, x_hbm, out_hbm,
             tabs_v, rl_v, a1_v, in0, in1, ob0, ob1, si0, si1, so0, so1):
        wid = lax.axis_index("s") * 2 + lax.axis_index("c")

        pltpu.sync_copy(tabs_hbm, tabs_v)
        pltpu.sync_copy(rowloc_hbm, rl_v)

        def p1_body(k, carry):
            chunk = wid + NWORKERS * k

            @pl.when(chunk < NCHUNKS)
            def _():
                for v in range(VPC):
                    g = v // (GROUP_W // 16)
                    t = (v % (GROUP_W // 16)) * 16
                    rbase = chunk * CHUNK_ROWS + g * 4
                    row = rl_v[pl.ds(t, 16)] + rbase.astype(jnp.float32)
                    q = (row + 0.5) * (1.0 / FEAT_W)
                    gy = q.astype(jnp.int32).astype(jnp.float32)
                    gx = row - FEAT_W * gy
                    a1 = (tabs_v[pl.ds(GROUP_W + t, 16)]
                          + 16.0 * (gx * tabs_v[pl.ds(3 * GROUP_W + t, 16)]
                                    + gy * tabs_v[pl.ds(4 * GROUP_W + t, 16)]))
                    a1_v[pl.ds(k * CHUNK_W + v * 16, 16)] = a1
            return carry

        lax.fori_loop(0, MAXK, p1_body, 0)

        def xoff(m):
            b = m // MAXK
            k = m % MAXK
            return b * PLANE + (wid + NWORKERS * k) * CHUNK_W

        def mvalid(m):
            return (wid + NWORKERS * (m % MAXK)) < NCHUNKS

        def start_in(m, buf, sem):
            pltpu.make_async_copy(
                x_hbm.at[pl.ds(xoff(m), CHUNK_W)], buf, sem).start()

        def wait_in(m, buf, sem):
            pltpu.make_async_copy(
                x_hbm.at[pl.ds(xoff(m), CHUNK_W)], buf, sem).wait()

        def start_out(m, buf, sem):
            pltpu.make_async_copy(
                buf, out_hbm.at[pl.ds(xoff(m), CHUNK_W)], sem).start()

        def wait_out(m, buf, sem):
            pltpu.make_async_copy(
                buf, out_hbm.at[pl.ds(xoff(m), CHUNK_W)], sem).wait()

        start_in(0, in0, si0)

        def p2_body(jj, carry):
            for p in range(2):
                inbuf, si = (in0, si0) if p == 0 else (in1, si1)
                obuf, so = (ob0, so0) if p == 0 else (ob1, so1)
                nxt, sn = (in1, si1) if p == 0 else (in0, si0)
                m = 2 * jj + p

                @pl.when((m + 1 < NSTEPS) & mvalid(m + 1))
                def _():
                    start_in(m + 1, nxt, sn)

                @pl.when((m >= 2) & mvalid(m - 2))
                def _():
                    wait_out(m - 2, obuf, so)

                @pl.when(mvalid(m))
                def _():
                    wait_in(m, inbuf, si)
                    k = m % MAXK
                    for v in range(VPC):
                        t = (v % (GROUP_W // 16)) * 16
                        u = (inbuf[pl.ds(v * 16, 16)] * tabs_v[pl.ds(t, 16)]
                             + a1_v[pl.ds(k * CHUNK_W + v * 16, 16)])
                        e = jnp.exp(u)
                        msk = tabs_v[pl.ds(2 * GROUP_W + t, 16)]
                        obuf[pl.ds(v * 16, 16)] = u + msk * (e - u)
                    start_out(m, obuf, so)
            return carry

        lax.fori_loop(0, NSTEPS // 2, p2_body, 0)

        @pl.when(mvalid(NSTEPS - 2))
        def _():
            wait_out(NSTEPS - 2, ob0, so0)

        @pl.when(mvalid(NSTEPS - 1))
        def _():
            wait_out(NSTEPS - 1, ob1, so1)

    return body(tabs, rowloc, xflat)


def kernel(cls, prob, bbox_2d, bbox_3d, anchors, bbox_means, bbox_stds):
    del cls, prob, bbox_2d
    f32 = jnp.float32

    aidx = jnp.arange(ROWW, dtype=jnp.int32) // NCH
    cidx = jnp.arange(ROWW, dtype=jnp.int32) % NCH

    aw = anchors[:, 2] - anchors[:, 0] + 1.0
    ah = anchors[:, 3] - anchors[:, 1] + 1.0
    wj = aw[aidx]
    hj = ah[aidx]
    std_c = bbox_stds[0, 4:11][cidx]
    mean_c = bbox_means[0, 4:11][cidx]
    x1j = anchors[aidx, 0]
    y1j = anchors[aidx, 1]
    z3j = anchors[aidx, 4]
    dimj = jnp.where(cidx == 3, anchors[aidx, 5],
                     jnp.where(cidx == 4, anchors[aidx, 6], anchors[aidx, 7]))
    ry3j = anchors[aidx, 8]

    isexp = ((cidx >= 3) & (cidx <= 5)).astype(f32)
    lmul = jnp.where(cidx == 0, wj, jnp.where(cidx == 1, hj, 1.0))
    ladd = jnp.where(cidx == 0, x1j + 0.5 * wj,
                     jnp.where(cidx == 1, y1j + 0.5 * hj,
                               jnp.where(cidx == 2, z3j,
                                         jnp.where(cidx == 6, ry3j, 0.0))))
    m1 = jnp.where(isexp > 0, std_c, std_c * lmul)
    base = jnp.where(isexp > 0, mean_c + jnp.log(dimj),
                     mean_c * lmul + ladd)
    isc0 = (cidx == 0).astype(f32)
    isc1 = (cidx == 1).astype(f32)

    tabs = jnp.concatenate([
        jnp.tile(m1, 4), jnp.tile(base, 4), jnp.tile(isexp, 4),
        jnp.tile(isc0, 4), jnp.tile(isc1, 4),
    ]).astype(f32)
    rowloc = (jnp.arange(GROUP_W, dtype=jnp.int32) // ROWW).astype(f32)

    xflat = bbox_3d.reshape(-1).astype(f32)
    out = _sc_denorm(tabs, rowloc, xflat)
    return out.reshape(BATCH, HW * NUM_A, NCH)
